# baseline (speedup 1.0000x reference)
import functools

import jax
import jax.numpy as jnp
from jax.experimental import pallas as pl


def _row_norm_t(residual):
    yt = jnp.transpose(residual)
    yt = yt * yt
    s = yt[0:8, :]
    for i in range(1, 32):
        s = s + yt[8 * i:8 * i + 8, :]
    w = 8
    while w > 1:
        w //= 2
        s = s[:w, :] + s[w:2 * w, :]
    return jnp.transpose(s)


def _argmin_row(d2):
    bb, k = d2.shape
    w = k // 2
    lanes = jax.lax.broadcasted_iota(jnp.int32, (bb, w), 1)
    a, b = d2[:, :w], d2[:, w:]
    mask = a <= b
    val = jnp.where(mask, a, b)
    pos = jnp.where(mask, lanes, lanes + w)
    while w > 1:
        w //= 2
        a, b = val[:, :w], val[:, w:]
        mask = a <= b
        val = jnp.where(mask, a, b)
        pos = jnp.where(mask, pos[:, :w], pos[:, w:])
    return pos


def _rqvae_body(x_ref, ew_ref, eb_ref, cb_ref, cbh_ref, cbm_ref, cbl_ref,
                cn_ref, ids_ref):
    r = jnp.dot(x_ref[...], ew_ref[...],
                preferred_element_type=jnp.float32) + eb_ref[...]
    num_layers, k, c = cb_ref.shape
    bb = r.shape[0]
    iotaf = jax.lax.broadcasted_iota(jnp.int32, (bb, k), 1).astype(jnp.float32)
    residual = r
    cols = []
    for l in range(num_layers):
        cbh = cbh_ref[l]
        rn = _row_norm_t(residual)
        d2 = (rn - 2.0 * jax.lax.dot_general(
            residual, cb_ref[l], (((1,), (1,)), ((), ())),
            preferred_element_type=jnp.float32)) + cn_ref[l:l + 1, :]
        m = jnp.min(d2, axis=1, keepdims=True)
        idxf = jnp.min(jnp.where(d2 == m, iotaf, float(k)), axis=1,
                       keepdims=True)
        idx = idxf.astype(jnp.int32)
        onehot = (idxf == iotaf).astype(jnp.bfloat16)
        dn = (((1,), (0,)), ((), ()))
        quant = ((jax.lax.dot_general(onehot, cbh, dn,
                                      preferred_element_type=jnp.float32)
                  + jax.lax.dot_general(onehot, cbm_ref[l], dn,
                                        preferred_element_type=jnp.float32))
                 + jax.lax.dot_general(onehot, cbl_ref[l], dn,
                                       preferred_element_type=jnp.float32))
        residual = residual - quant
        cols.append(idx)
    ids_ref[...] = jnp.concatenate(cols, axis=1).astype(jnp.int32)


@functools.partial(jax.jit, static_argnames=("block_b",))
def _run(x, enc_W, enc_b2, codebooks, dec_W, dec_b, block_b=512):
    b, d_in = x.shape
    num_layers, k, c = codebooks.shape
    cn = jnp.stack([jnp.sum(codebooks[i] * codebooks[i], axis=1)
                    for i in range(num_layers)])
    cb_hi = jax.lax.optimization_barrier(codebooks.astype(jnp.bfloat16))
    rem = jax.lax.optimization_barrier(
        codebooks - cb_hi.astype(jnp.float32))
    cb_mid = jax.lax.optimization_barrier(rem.astype(jnp.bfloat16))
    cb_lo = jax.lax.optimization_barrier(
        (rem - cb_mid.astype(jnp.float32)).astype(jnp.bfloat16))
    grid = (b // block_b,)
    ids = pl.pallas_call(
        _rqvae_body,
        grid=grid,
        in_specs=[
            pl.BlockSpec((block_b, d_in), lambda i: (i, 0)),
            pl.BlockSpec((d_in, c), lambda i: (0, 0)),
            pl.BlockSpec((1, c), lambda i: (0, 0)),
            pl.BlockSpec((num_layers, k, c), lambda i: (0, 0, 0)),
            pl.BlockSpec((num_layers, k, c), lambda i: (0, 0, 0)),
            pl.BlockSpec((num_layers, k, c), lambda i: (0, 0, 0)),
            pl.BlockSpec((num_layers, k, c), lambda i: (0, 0, 0)),
            pl.BlockSpec((num_layers, k), lambda i: (0, 0)),
        ],
        out_specs=pl.BlockSpec((block_b, num_layers), lambda i: (i, 0)),
        out_shape=jax.ShapeDtypeStruct((b, num_layers), jnp.int32),
    )(x, enc_W, enc_b2, codebooks, cb_hi, cb_mid, cb_lo, cn)
    recon = ids.astype(jnp.float32) @ dec_W + dec_b
    return recon, ids


def kernel(dense_content_embedding, enc_W, enc_b, codebooks, dec_W, dec_b):
    enc_b2 = enc_b.reshape(1, -1)
    return _run(dense_content_embedding, enc_W, enc_b2, codebooks, dec_W,
                dec_b)


# speedup vs baseline: 1.0804x; 1.0804x over previous
import functools

import jax
import jax.numpy as jnp
from jax.experimental import pallas as pl


def _row_norm_t(residual):
    yt = jnp.transpose(residual)
    yt = yt * yt
    s = yt[0:8, :]
    for i in range(1, 32):
        s = s + yt[8 * i:8 * i + 8, :]
    w = 8
    while w > 1:
        w //= 2
        s = s[:w, :] + s[w:2 * w, :]
    return jnp.transpose(s)


def _argmin_row(d2):
    bb, k = d2.shape
    w = k // 2
    lanes = jax.lax.broadcasted_iota(jnp.int32, (bb, w), 1)
    a, b = d2[:, :w], d2[:, w:]
    mask = a <= b
    val = jnp.where(mask, a, b)
    pos = jnp.where(mask, lanes, lanes + w)
    while w > 1:
        w //= 2
        a, b = val[:, :w], val[:, w:]
        mask = a <= b
        val = jnp.where(mask, a, b)
        pos = jnp.where(mask, pos[:, :w], pos[:, w:])
    return pos


def _rqvae_body(x_ref, ew_ref, eb_ref, cbh2_ref, cbh_ref, cbm_ref, cbl_ref,
                cn_ref, ids_ref):
    r = jnp.dot(x_ref[...], ew_ref[...],
                preferred_element_type=jnp.float32) + eb_ref[...]
    num_layers, k, c = cbh_ref.shape
    bb = r.shape[0]
    iotaf = jax.lax.broadcasted_iota(jnp.int32, (bb, k), 1).astype(jnp.float32)
    residual = r
    cols = []
    for l in range(num_layers):
        cbh = cbh_ref[l]
        rn = _row_norm_t(residual)
        d2 = (rn - jax.lax.dot_general(
            residual, cbh2_ref[l], (((1,), (1,)), ((), ())),
            preferred_element_type=jnp.float32)) + cn_ref[l:l + 1, :]
        m = jnp.min(d2, axis=1, keepdims=True)
        idxf = jnp.min(jnp.where(d2 == m, iotaf, float(k)), axis=1,
                       keepdims=True)
        idx = idxf.astype(jnp.int32)
        onehot = (idxf == iotaf).astype(jnp.bfloat16)
        dn = (((1,), (0,)), ((), ()))
        quant = ((jax.lax.dot_general(onehot, cbh, dn,
                                      preferred_element_type=jnp.float32)
                  + jax.lax.dot_general(onehot, cbm_ref[l], dn,
                                        preferred_element_type=jnp.float32))
                 + jax.lax.dot_general(onehot, cbl_ref[l], dn,
                                       preferred_element_type=jnp.float32))
        residual = residual - quant
        cols.append(idx)
    ids_ref[...] = jnp.concatenate(cols, axis=1).astype(jnp.int32)


@functools.partial(jax.jit, static_argnames=("block_b",))
def _run(x, enc_W, enc_b2, codebooks, dec_W, dec_b, block_b=1024):
    b, d_in = x.shape
    num_layers, k, c = codebooks.shape
    cn = jnp.stack([jnp.sum(codebooks[i] * codebooks[i], axis=1)
                    for i in range(num_layers)])
    cb_hi = jax.lax.optimization_barrier(codebooks.astype(jnp.bfloat16))
    rem = jax.lax.optimization_barrier(
        codebooks - cb_hi.astype(jnp.float32))
    cb_mid = jax.lax.optimization_barrier(rem.astype(jnp.bfloat16))
    cb_lo = jax.lax.optimization_barrier(
        (rem - cb_mid.astype(jnp.float32)).astype(jnp.bfloat16))
    cb_hi2 = cb_hi * jnp.bfloat16(2.0)
    grid = (b // block_b,)
    ids = pl.pallas_call(
        _rqvae_body,
        grid=grid,
        in_specs=[
            pl.BlockSpec((block_b, d_in), lambda i: (i, 0)),
            pl.BlockSpec((d_in, c), lambda i: (0, 0)),
            pl.BlockSpec((1, c), lambda i: (0, 0)),
            pl.BlockSpec((num_layers, k, c), lambda i: (0, 0, 0)),
            pl.BlockSpec((num_layers, k, c), lambda i: (0, 0, 0)),
            pl.BlockSpec((num_layers, k, c), lambda i: (0, 0, 0)),
            pl.BlockSpec((num_layers, k, c), lambda i: (0, 0, 0)),
            pl.BlockSpec((num_layers, k), lambda i: (0, 0)),
        ],
        out_specs=pl.BlockSpec((block_b, num_layers), lambda i: (i, 0)),
        out_shape=jax.ShapeDtypeStruct((b, num_layers), jnp.int32),
    )(x, enc_W, enc_b2, cb_hi2, cb_hi, cb_mid, cb_lo, cn)
    recon = ids.astype(jnp.float32) @ dec_W + dec_b
    return recon, ids


def kernel(dense_content_embedding, enc_W, enc_b, codebooks, dec_W, dec_b):
    enc_b2 = enc_b.reshape(1, -1)
    return _run(dense_content_embedding, enc_W, enc_b2, codebooks, dec_W,
                dec_b)


# speedup vs baseline: 1.1270x; 1.0431x over previous
import functools

import jax
import jax.numpy as jnp
from jax.experimental import pallas as pl
from jax.experimental.pallas import tpu as pltpu


def _row_norm_t(residual):
    yt = jnp.transpose(residual)
    yt = yt * yt
    s = yt[0:8, :]
    for i in range(1, 32):
        s = s + yt[8 * i:8 * i + 8, :]
    w = 8
    while w > 1:
        w //= 2
        s = s[:w, :] + s[w:2 * w, :]
    return jnp.transpose(s)


def _argmin_row(d2):
    bb, k = d2.shape
    w = k // 2
    lanes = jax.lax.broadcasted_iota(jnp.int32, (bb, w), 1)
    a, b = d2[:, :w], d2[:, w:]
    mask = a <= b
    val = jnp.where(mask, a, b)
    pos = jnp.where(mask, lanes, lanes + w)
    while w > 1:
        w //= 2
        a, b = val[:, :w], val[:, w:]
        mask = a <= b
        val = jnp.where(mask, a, b)
        pos = jnp.where(mask, pos[:, :w], pos[:, w:])
    return pos


def _rqvae_body(x_ref, ew_ref, eb_ref, cb_ref, cn_ref, ids_ref,
                cbh2_ref, cbh_ref, cbm_ref, cbl_ref):
    num_layers, k, c = cb_ref.shape

    @pl.when(pl.program_id(0) == 0)
    def _split():
        cbf = cb_ref[...]
        hi = cbf.astype(jnp.bfloat16)
        cbh_ref[...] = hi
        cbh2_ref[...] = hi * jnp.bfloat16(2.0)
        rem = cbf - hi.astype(jnp.float32)
        mid = rem.astype(jnp.bfloat16)
        cbm_ref[...] = mid
        cbl_ref[...] = (rem - mid.astype(jnp.float32)).astype(jnp.bfloat16)

    r = jnp.dot(x_ref[...], ew_ref[...],
                preferred_element_type=jnp.float32) + eb_ref[...]
    bb = r.shape[0]
    iotaf = jax.lax.broadcasted_iota(jnp.int32, (bb, k), 1).astype(jnp.float32)
    residual = r
    cols = []
    for l in range(num_layers):
        cbh = cbh_ref[l]
        rn = _row_norm_t(residual)
        d2 = (rn - jax.lax.dot_general(
            residual, cbh2_ref[l], (((1,), (1,)), ((), ())),
            preferred_element_type=jnp.float32)) + cn_ref[l:l + 1, :]
        m = jnp.min(d2, axis=1, keepdims=True)
        idxf = jnp.min(jnp.where(d2 == m, iotaf, float(k)), axis=1,
                       keepdims=True)
        idx = idxf.astype(jnp.int32)
        onehot = (idxf == iotaf).astype(jnp.bfloat16)
        dn = (((1,), (0,)), ((), ()))
        quant = ((jax.lax.dot_general(onehot, cbh, dn,
                                      preferred_element_type=jnp.float32)
                  + jax.lax.dot_general(onehot, cbm_ref[l], dn,
                                        preferred_element_type=jnp.float32))
                 + jax.lax.dot_general(onehot, cbl_ref[l], dn,
                                       preferred_element_type=jnp.float32))
        residual = residual - quant
        cols.append(idx)
    ids_ref[...] = jnp.concatenate(cols, axis=1).astype(jnp.int32)


@functools.partial(jax.jit, static_argnames=("block_b",))
def _run(x, enc_W, enc_b2, codebooks, dec_W, dec_b, block_b=1024):
    b, d_in = x.shape
    num_layers, k, c = codebooks.shape
    cn = jnp.stack([jnp.sum(codebooks[i] * codebooks[i], axis=1)
                    for i in range(num_layers)])
    grid = (b // block_b,)
    ids = pl.pallas_call(
        _rqvae_body,
        grid=grid,
        in_specs=[
            pl.BlockSpec((block_b, d_in), lambda i: (i, 0)),
            pl.BlockSpec((d_in, c), lambda i: (0, 0)),
            pl.BlockSpec((1, c), lambda i: (0, 0)),
            pl.BlockSpec((num_layers, k, c), lambda i: (0, 0, 0)),
            pl.BlockSpec((num_layers, k), lambda i: (0, 0)),
        ],
        out_specs=pl.BlockSpec((block_b, num_layers), lambda i: (i, 0)),
        out_shape=jax.ShapeDtypeStruct((b, num_layers), jnp.int32),
        scratch_shapes=[pltpu.VMEM((num_layers, k, c), jnp.bfloat16)
                        for _ in range(4)],
    )(x, enc_W, enc_b2, codebooks, cn)
    recon = ids.astype(jnp.float32) @ dec_W + dec_b
    return recon, ids


def kernel(dense_content_embedding, enc_W, enc_b, codebooks, dec_W, dec_b):
    enc_b2 = enc_b.reshape(1, -1)
    return _run(dense_content_embedding, enc_W, enc_b2, codebooks, dec_W,
                dec_b)
